# baseline (speedup 1.0000x reference)
import jax
import jax.numpy as jnp
from jax.experimental import pallas as pl
from jax.experimental.pallas import tpu as pltpu

_B, _T, _D = 8, 2048, 256
_K = 1024
_CC = 0.25
_M = _B * _T
_TM = 1024
_GRID = _M // _TM
_TILES_PER_B = _T // _TM


def _vq_body(len_ref, x_ref, w_ref, xsq_ref, wsq_ref,
             enc_ref, qst_ref, loss_ref, perp_ref,
             counts_ref, lsum_ref, nsum_ref):
    pid = pl.program_id(0)

    @pl.when(pid == 0)
    def _init():
        counts_ref[...] = jnp.zeros_like(counts_ref)
        lsum_ref[0, 0] = 0.0
        nsum_ref[0, 0] = 0.0

    x = x_ref[...]
    w = w_ref[...]

    xw = jax.lax.dot_general(
        x, w, (((1,), (1,)), ((), ())),
        preferred_element_type=jnp.float32)
    d = (xsq_ref[...] + wsq_ref[...]) - 2.0 * xw

    m = jnp.min(d, axis=1, keepdims=True)
    kiota = jax.lax.broadcasted_iota(jnp.int32, (_TM, _K), 1)
    idx = jnp.min(jnp.where(d == m, kiota, _K), axis=1, keepdims=True)
    enc = (kiota == idx).astype(jnp.float32)
    enc_ref[...] = enc
    counts_ref[...] += jnp.sum(enc, axis=0, keepdims=True)

    q = jax.lax.dot_general(
        enc.astype(jnp.bfloat16), w.astype(jnp.bfloat16),
        (((1,), (0,)), ((), ())),
        preferred_element_type=jnp.float32)
    qst_ref[...] = x + (q - x)

    length = len_ref[pid // _TILES_PER_B] // 2
    t = (pid % _TILES_PER_B) * _TM + jax.lax.broadcasted_iota(
        jnp.int32, (_TM, 1), 0)
    valid = (t < length).astype(jnp.float32)
    lsum_ref[0, 0] += jnp.sum(valid * m)
    nsum_ref[0, 0] += jnp.sum(valid)

    @pl.when(pid == _GRID - 1)
    def _fin():
        loss = (1.0 + _CC) * (lsum_ref[0, 0] / _D) / nsum_ref[0, 0]
        loss_ref[...] = jnp.full((1, 1), loss, jnp.float32)
        avg = counts_ref[...] / jnp.float32(_M)
        perp = jnp.exp(-jnp.sum(avg * jnp.log(avg + 1e-10)))
        perp_ref[...] = jnp.full((1, 1), perp, jnp.float32)


def kernel(inputs, data_len, weight):
    flat = inputs.reshape(-1, _D)
    xsq = jnp.sum(flat ** 2, axis=1, keepdims=True)
    wsq = jnp.sum(weight ** 2, axis=1)[None, :]

    grid_spec = pltpu.PrefetchScalarGridSpec(
        num_scalar_prefetch=1,
        grid=(_GRID,),
        in_specs=[
            pl.BlockSpec((_TM, _D), lambda i, *_: (i, 0)),
            pl.BlockSpec((_K, _D), lambda i, *_: (0, 0)),
            pl.BlockSpec((_TM, 1), lambda i, *_: (i, 0)),
            pl.BlockSpec((1, _K), lambda i, *_: (0, 0)),
        ],
        out_specs=[
            pl.BlockSpec((_TM, _K), lambda i, *_: (i, 0)),
            pl.BlockSpec((_TM, _D), lambda i, *_: (i, 0)),
            pl.BlockSpec((1, 1), lambda i, *_: (0, 0)),
            pl.BlockSpec((1, 1), lambda i, *_: (0, 0)),
        ],
        scratch_shapes=[
            pltpu.VMEM((1, _K), jnp.float32),
            pltpu.SMEM((1, 1), jnp.float32),
            pltpu.SMEM((1, 1), jnp.float32),
        ],
    )
    enc, qst, loss, perp = pl.pallas_call(
        _vq_body,
        grid_spec=grid_spec,
        out_shape=[
            jax.ShapeDtypeStruct((_M, _K), jnp.float32),
            jax.ShapeDtypeStruct((_M, _D), jnp.float32),
            jax.ShapeDtypeStruct((1, 1), jnp.float32),
            jax.ShapeDtypeStruct((1, 1), jnp.float32),
        ],
    )(data_len, flat, weight, xsq, wsq)
    return (loss.reshape(()), qst.reshape(inputs.shape),
            perp.reshape(()), enc)


# speedup vs baseline: 1.5184x; 1.5184x over previous
import jax
import jax.numpy as jnp
from jax.experimental import pallas as pl
from jax.experimental.pallas import tpu as pltpu

_B, _T, _D = 8, 2048, 256
_K = 1024
_CC = 0.25
_M = _B * _T
_TM = 2048
_GRID = _M // _TM
_TILES_PER_B = _T // _TM


def _vq_body(len_ref, x_ref, w_ref,
             enc_ref, qst_ref, loss_ref, perp_ref,
             counts_ref, wsq_ref, lsum_ref, nsum_ref):
    pid = pl.program_id(0)

    x = x_ref[...]
    w = w_ref[...]

    @pl.when(pid == 0)
    def _init():
        counts_ref[...] = jnp.zeros_like(counts_ref)
        lsum_ref[0, 0] = 0.0
        nsum_ref[0, 0] = 0.0
        wsq_ref[...] = jnp.sum(w * w, axis=1)[None, :]

    xw = jax.lax.dot_general(
        x, w, (((1,), (1,)), ((), ())),
        preferred_element_type=jnp.float32)

    xsq = jnp.sum(x * x, axis=1, keepdims=True)
    liota = jax.lax.broadcasted_iota(
        jnp.int32, (_TM, 128), 1).astype(jnp.float32)
    m_run = jnp.full((_TM, 128), jnp.inf, jnp.float32)
    i_run = jnp.zeros((_TM, 128), jnp.float32)
    for j in range(_K // 128):
        dj = (xsq + wsq_ref[:, j * 128:(j + 1) * 128]) \
            - 2.0 * xw[:, j * 128:(j + 1) * 128]
        i_run = jnp.where(dj < m_run, liota + (j * 128.0), i_run)
        m_run = jnp.minimum(dj, m_run)
    m = jnp.min(m_run, axis=1, keepdims=True)
    idx = jnp.min(jnp.where(m_run == m, i_run, jnp.float32(_K)),
                  axis=1, keepdims=True)
    kiota = jax.lax.broadcasted_iota(jnp.int32, (_TM, _K), 1)
    enc = (kiota == idx.astype(jnp.int32)).astype(jnp.float32)
    enc_ref[...] = enc
    counts_ref[...] += jnp.sum(enc, axis=0, keepdims=True)

    q = jax.lax.dot_general(
        enc.astype(jnp.bfloat16), w.astype(jnp.bfloat16),
        (((1,), (0,)), ((), ())),
        preferred_element_type=jnp.float32)
    qst_ref[...] = x + (q - x)

    length = len_ref[pid // _TILES_PER_B] // 2
    t = (pid % _TILES_PER_B) * _TM + jax.lax.broadcasted_iota(
        jnp.int32, (_TM, 1), 0)
    valid = (t < length).astype(jnp.float32)
    lsum_ref[0, 0] += jnp.sum(valid * m)
    nsum_ref[0, 0] += jnp.sum(valid)

    @pl.when(pid == _GRID - 1)
    def _fin():
        loss = (1.0 + _CC) * (lsum_ref[0, 0] / _D) / nsum_ref[0, 0]
        loss_ref[...] = jnp.full((1, 1), loss, jnp.float32)
        avg = counts_ref[...] / jnp.float32(_M)
        perp = jnp.exp(-jnp.sum(avg * jnp.log(avg + 1e-10)))
        perp_ref[...] = jnp.full((1, 1), perp, jnp.float32)


def kernel(inputs, data_len, weight):
    flat = inputs.reshape(-1, _D)

    grid_spec = pltpu.PrefetchScalarGridSpec(
        num_scalar_prefetch=1,
        grid=(_GRID,),
        in_specs=[
            pl.BlockSpec((_TM, _D), lambda i, *_: (i, 0)),
            pl.BlockSpec((_K, _D), lambda i, *_: (0, 0)),
        ],
        out_specs=[
            pl.BlockSpec((_TM, _K), lambda i, *_: (i, 0)),
            pl.BlockSpec((_TM, _D), lambda i, *_: (i, 0)),
            pl.BlockSpec((1, 1), lambda i, *_: (0, 0)),
            pl.BlockSpec((1, 1), lambda i, *_: (0, 0)),
        ],
        scratch_shapes=[
            pltpu.VMEM((1, _K), jnp.float32),
            pltpu.VMEM((1, _K), jnp.float32),
            pltpu.SMEM((1, 1), jnp.float32),
            pltpu.SMEM((1, 1), jnp.float32),
        ],
    )
    enc, qst, loss, perp = pl.pallas_call(
        _vq_body,
        grid_spec=grid_spec,
        out_shape=[
            jax.ShapeDtypeStruct((_M, _K), jnp.float32),
            jax.ShapeDtypeStruct((_M, _D), jnp.float32),
            jax.ShapeDtypeStruct((1, 1), jnp.float32),
            jax.ShapeDtypeStruct((1, 1), jnp.float32),
        ],
    )(data_len, flat, weight)
    return (loss.reshape(()), qst.reshape(inputs.shape),
            perp.reshape(()), enc)


# speedup vs baseline: 1.5236x; 1.0034x over previous
import jax
import jax.numpy as jnp
from jax.experimental import pallas as pl
from jax.experimental.pallas import tpu as pltpu

_B, _T, _D = 8, 2048, 256
_K = 1024
_CC = 0.25
_M = _B * _T
_TM = 2048
_GRID = _M // _TM
_TILES_PER_B = _T // _TM


def _vq_body(len_ref, x_ref, w_ref,
             enc_ref, qst_ref, loss_ref, perp_ref,
             counts_ref, wsq_ref, lsum_ref, nsum_ref):
    pid = pl.program_id(0)

    x = x_ref[...]
    w = w_ref[...]

    @pl.when(pid == 0)
    def _init():
        counts_ref[...] = jnp.zeros_like(counts_ref)
        lsum_ref[0, 0] = 0.0
        nsum_ref[0, 0] = 0.0
        wsq_ref[...] = jnp.sum(w * w, axis=1)[None, :]

    xw = jax.lax.dot_general(
        x, w, (((1,), (1,)), ((), ())),
        preferred_element_type=jnp.float32)

    xsq = jnp.sum(x * x, axis=1, keepdims=True)
    liota = jax.lax.broadcasted_iota(
        jnp.int32, (_TM, 128), 1).astype(jnp.float32)
    m_run = jnp.full((_TM, 128), jnp.inf, jnp.float32)
    i_run = jnp.zeros((_TM, 128), jnp.float32)
    for j in range(_K // 128):
        dj = (xsq + wsq_ref[:, j * 128:(j + 1) * 128]) \
            - 2.0 * xw[:, j * 128:(j + 1) * 128]
        i_run = jnp.where(dj < m_run, liota + (j * 128.0), i_run)
        m_run = jnp.minimum(dj, m_run)
    m = jnp.min(m_run, axis=1, keepdims=True)
    idx = jnp.min(jnp.where(m_run == m, i_run, jnp.float32(_K)),
                  axis=1, keepdims=True)
    kiota = jax.lax.broadcasted_iota(jnp.int32, (_TM, _K), 1)
    enc = (kiota == idx.astype(jnp.int32)).astype(jnp.float32)
    enc_ref[...] = enc
    counts_ref[...] += jnp.sum(enc, axis=0, keepdims=True)
    encb = enc.astype(jnp.bfloat16)

    q = jax.lax.dot_general(
        encb, w.astype(jnp.bfloat16),
        (((1,), (0,)), ((), ())),
        preferred_element_type=jnp.float32)
    qst_ref[...] = q

    length = len_ref[pid // _TILES_PER_B] // 2
    t = (pid % _TILES_PER_B) * _TM + jax.lax.broadcasted_iota(
        jnp.int32, (_TM, 1), 0)
    valid = (t < length).astype(jnp.float32)
    lsum_ref[0, 0] += jnp.sum(valid * m)
    nsum_ref[0, 0] += jnp.sum(valid)

    @pl.when(pid == _GRID - 1)
    def _fin():
        loss = (1.0 + _CC) * (lsum_ref[0, 0] / _D) / nsum_ref[0, 0]
        loss_ref[...] = jnp.full((1, 1), loss, jnp.float32)
        avg = counts_ref[...] / jnp.float32(_M)
        perp = jnp.exp(-jnp.sum(avg * jnp.log(avg + 1e-10)))
        perp_ref[...] = jnp.full((1, 1), perp, jnp.float32)


def kernel(inputs, data_len, weight):
    flat = inputs.reshape(-1, _D)

    grid_spec = pltpu.PrefetchScalarGridSpec(
        num_scalar_prefetch=1,
        grid=(_GRID,),
        in_specs=[
            pl.BlockSpec((_TM, _D), lambda i, *_: (i, 0)),
            pl.BlockSpec((_K, _D), lambda i, *_: (0, 0)),
        ],
        out_specs=[
            pl.BlockSpec((_TM, _K), lambda i, *_: (i, 0)),
            pl.BlockSpec((_TM, _D), lambda i, *_: (i, 0)),
            pl.BlockSpec((1, 1), lambda i, *_: (0, 0)),
            pl.BlockSpec((1, 1), lambda i, *_: (0, 0)),
        ],
        scratch_shapes=[
            pltpu.VMEM((1, _K), jnp.float32),
            pltpu.VMEM((1, _K), jnp.float32),
            pltpu.SMEM((1, 1), jnp.float32),
            pltpu.SMEM((1, 1), jnp.float32),
        ],
    )
    enc, qst, loss, perp = pl.pallas_call(
        _vq_body,
        grid_spec=grid_spec,
        out_shape=[
            jax.ShapeDtypeStruct((_M, _K), jnp.float32),
            jax.ShapeDtypeStruct((_M, _D), jnp.float32),
            jax.ShapeDtypeStruct((1, 1), jnp.float32),
            jax.ShapeDtypeStruct((1, 1), jnp.float32),
        ],
    )(data_len, flat, weight)
    return (loss.reshape(()), qst.reshape(inputs.shape),
            perp.reshape(()), enc)
